# speedup vs baseline: 121.4360x; 121.4360x over previous
import jax
import jax.numpy as jnp
from jax.experimental import pallas as pl

_TOP_P = 0.98
_ROWS_PER_STEP = 8
_BISECT_ITERS = 28


def _topp_body(nl_ref, br_ref, dl_ref, out_ref):
    x = nl_ref[...]
    b = br_ref[...]
    d = dl_ref[...]

    ln = jnp.sqrt(jnp.sum(x * x, axis=1, keepdims=True))
    bn = jnp.sqrt(jnp.sum(b * b, axis=1, keepdims=True))
    scale = jnp.where(bn > 1e-12, ln / jnp.maximum(bn, 1e-12), 1.0)

    v = x + scale * d
    m = jnp.max(v, axis=1, keepdims=True)
    p = jnp.exp(v - m)
    target = _TOP_P * jnp.sum(p, axis=1, keepdims=True)

    lo = jnp.min(v, axis=1, keepdims=True) - 1.0
    hi = m

    def body(_, carry):
        lo, hi = carry
        mid = 0.5 * (lo + hi)
        mass = jnp.sum(jnp.where(v > mid, p, 0.0), axis=1, keepdims=True)
        go_down = mass <= target
        return jnp.where(go_down, lo, mid), jnp.where(go_down, mid, hi)

    lo, hi = jax.lax.fori_loop(0, _BISECT_ITERS, body, (lo, hi))

    out_ref[...] = jnp.where(v > lo, v, -jnp.inf)


def kernel(next_logits, bias_row, delta):
    B, V = next_logits.shape
    grid = (B // _ROWS_PER_STEP,)
    spec = pl.BlockSpec((_ROWS_PER_STEP, V), lambda i: (i, 0))
    return pl.pallas_call(
        _topp_body,
        grid=grid,
        in_specs=[spec, spec, spec],
        out_specs=spec,
        out_shape=jax.ShapeDtypeStruct((B, V), next_logits.dtype),
    )(next_logits, bias_row, delta)
